# baseline (speedup 1.0000x reference)
import functools

import jax
import jax.numpy as jnp
import numpy as np
from jax import lax
from jax.experimental import pallas as pl
from jax.experimental.pallas import tpu as pltpu
from jax.experimental.pallas import tpu_sc as plsc

N = 10000
E = 320000
F = 128
EPS = 1e-8

NC, NS = 2, 16
NW = NC * NS
EPW = E // NW
CB = 40
NBLK = EPW // CB
NQ = 5
CPQ = NBLK // NQ
N_PAD = 10240
RPT = N_PAD // NS

_ACC_PERM = np.concatenate(
    [np.concatenate([32 * m + 2 * np.arange(16),
                     32 * m + 2 * np.arange(16) + 1]) for m in range(4)])


def _mm_body(a_ref, w_ref, b_ref, o_ref):
    r = jnp.dot(a_ref[...].astype(jnp.bfloat16), w_ref[...].astype(jnp.bfloat16),
                preferred_element_type=jnp.float32) + b_ref[...]
    o_ref[...] = r.astype(jnp.bfloat16)


def _matmul_bias(a, w, b2d, blk):
    g = a.shape[0] // blk
    return pl.pallas_call(
        _mm_body,
        grid=(g,),
        in_specs=[
            pl.BlockSpec((blk, a.shape[1]), lambda i: (i, 0)),
            pl.BlockSpec(w.shape, lambda i: (0, 0)),
            pl.BlockSpec((1, b2d.shape[1]), lambda i: (0, 0)),
        ],
        out_specs=pl.BlockSpec((blk, w.shape[1]), lambda i: (i, 0)),
        out_shape=jax.ShapeDtypeStruct((a.shape[0], w.shape[1]), jnp.bfloat16),
    )(a, w, b2d)


def _sc_body(eaw_hbm, xw_hbm, col_hbm, row_hbm, acc_out, cnt_out,
             col_v, row_v, ea_v0, ea_v1, xg_v0, xg_v1, h_v0, h_v1,
             ones_v, zbs_v, acc_sh, cnt_sh,
             sem_ld0, sem_ld1, sem_sc0, sem_sc1):
    cid = lax.axis_index("c")
    sid = lax.axis_index("s")
    wid = cid * NS + sid

    ea_v = (ea_v0, ea_v1)
    xg_v = (xg_v0, xg_v1)
    h_v = (h_v0, h_v1)
    sem_ld = (sem_ld0, sem_ld1)
    sem_sc = (sem_sc0, sem_sc1)

    zf = jnp.zeros((16,), jnp.float32)
    of = jnp.ones((16,), jnp.float32)

    def zero_h(i, _):
        for k in range(F // 16):
            h_v0[i, pl.ds(k * 16, 16)] = zf
        return 0
    lax.fori_loop(0, CB, zero_h, 0)

    def fill_ones(i, _):
        ones_v[i, :] = of
        zbs_v[i, :] = zf
        return 0
    lax.fori_loop(0, CB, fill_ones, 0)

    for m in range(RPT // CB):
        pltpu.sync_copy(h_v0, acc_sh.at[pl.ds(sid * RPT + m * CB, CB)])
        pltpu.sync_copy(zbs_v, cnt_sh.at[pl.ds(sid * RPT + m * CB, CB)])
    plsc.subcore_barrier()

    def wait_load(p):
        pltpu.make_async_copy(eaw_hbm.at[pl.ds(0, CB)], ea_v[p], sem_ld[p]).wait()
        pltpu.make_async_copy(xw_hbm.at[col_v.at[0]], xg_v[p], sem_ld[p]).wait()

    def wait_scatter(p):
        pltpu.make_async_copy(h_v[p], acc_sh.at[row_v.at[0]], sem_sc[p]).wait()
        pltpu.make_async_copy(ones_v, cnt_sh.at[row_v.at[0]], sem_sc[p]).wait()

    for q in range(NQ):
        pltpu.sync_copy(col_hbm.at[wid, q], col_v)
        pltpu.sync_copy(row_hbm.at[wid, q], row_v)

        def start_load(j, p):
            e0 = wid * EPW + (q * CPQ + j) * CB
            pltpu.async_copy(eaw_hbm.at[pl.ds(e0, CB)], ea_v[p], sem_ld[p])
            pltpu.async_copy(xw_hbm.at[col_v.at[j]], xg_v[p], sem_ld[p])

        def process(j, m, p):
            wait_load(p)

            @pl.when(m > 0)
            def _():
                wait_scatter(p)

            def rowfn(i, _):
                for g in range(F // 32):
                    ae, be = plsc.unpack(
                        ea_v[p][i, pl.ds(32 * g, 32)],
                        format=plsc.PackFormat.INTERLEAVED)
                    ax, bx = plsc.unpack(
                        xg_v[p][i, pl.ds(32 * g, 32)],
                        format=plsc.PackFormat.INTERLEAVED)
                    h_v[p][i, pl.ds(32 * g, 16)] = jnp.maximum(ae + ax, 0.0)
                    h_v[p][i, pl.ds(32 * g + 16, 16)] = jnp.maximum(
                        be + bx, 0.0)
                return 0
            lax.fori_loop(0, CB, rowfn, 0)
            pltpu.async_copy(h_v[p], acc_sh.at[row_v.at[j]], sem_sc[p],
                             add=True)
            pltpu.async_copy(ones_v, cnt_sh.at[row_v.at[j]], sem_sc[p],
                             add=True)

        start_load(0, 0)

        def pair(m, _):
            start_load(2 * m + 1, 1)
            process(2 * m, m, 0)

            @pl.when(m < CPQ // 2 - 1)
            def _():
                start_load(2 * m + 2, 0)
            process(2 * m + 1, m, 1)
            return 0
        lax.fori_loop(0, CPQ // 2, pair, 0)
        wait_scatter(0)
        wait_scatter(1)

    plsc.subcore_barrier()
    pltpu.sync_copy(acc_sh.at[pl.ds(sid * RPT, RPT)],
                    acc_out.at[cid, pl.ds(sid * RPT, RPT)])
    pltpu.sync_copy(cnt_sh.at[pl.ds(sid * RPT, RPT)],
                    cnt_out.at[cid, pl.ds(sid * RPT, RPT)])


@functools.lru_cache(maxsize=1)
def _make_sc_call():
  return pl.kernel(
    _sc_body,
    out_type=[
        jax.ShapeDtypeStruct((NC, N_PAD, F), jnp.float32),
        jax.ShapeDtypeStruct((NC, N_PAD, 16), jnp.float32),
    ],
    mesh=plsc.VectorSubcoreMesh(core_axis_name="c", subcore_axis_name="s",
                                num_cores=NC, num_subcores=NS),
    scratch_types=(
        [pltpu.VMEM((CPQ, CB), jnp.int32)] * 2
        + [pltpu.VMEM((CB, F), jnp.bfloat16)] * 4
        + [pltpu.VMEM((CB, F), jnp.float32)] * 2
        + [pltpu.VMEM((CB, 16), jnp.float32),
           pltpu.VMEM((CB, 16), jnp.float32),
           pltpu.VMEM_SHARED((N_PAD, F), jnp.float32),
           pltpu.VMEM_SHARED((N_PAD, 16), jnp.float32)]
        + [pltpu.SemaphoreType.DMA] * 4
    ),
    compiler_params=pltpu.CompilerParams(use_tc_tiling_on_sc=False,
                                         needs_layout_passes=False),
  )


def _cross_ratio(m):
    a, b, c, d = m[0], m[1], m[2], m[3]
    d13 = jnp.sqrt(jnp.sum((a - c) ** 2)) + EPS
    d24 = jnp.sqrt(jnp.sum((b - d) ** 2)) + EPS
    d14 = jnp.sqrt(jnp.sum((a - d) ** 2)) + EPS
    d23 = jnp.sqrt(jnp.sum((b - c) ** 2)) + EPS
    return (d13 * d24) / (d14 * d23)


def _finish_body(acc_ref, cnt_ref, x_ref, wm2_ref, bm2_ref, wu1_ref, bu1_ref,
                 wu2_ref, bu2_ref, o_ref, scale_ref):
    i = pl.program_id(0)
    agg = acc_ref[0] + acc_ref[1]
    cnt = cnt_ref[0, :, 0] + cnt_ref[1, :, 0]
    denom = (cnt + EPS)[:, None]
    mean = (jnp.dot(agg, wm2_ref[...], preferred_element_type=jnp.float32)
            + cnt[:, None] * bm2_ref[...]) / denom
    xb = x_ref[...]
    u = jnp.maximum(
        jnp.dot(xb, wu1_ref[:F], preferred_element_type=jnp.float32)
        + jnp.dot(mean, wu1_ref[F:], preferred_element_type=jnp.float32)
        + bu1_ref[...], 0.0)
    y = jnp.dot(u, wu2_ref[...], preferred_element_type=jnp.float32) + bu2_ref[...]
    nrm = jnp.sqrt(jnp.sum(y * y, axis=1, keepdims=True))
    ynorm = y / (nrm + EPS)

    @pl.when(i == 0)
    def _():
        cr_t = _cross_ratio(xb)
        cr_n = _cross_ratio(ynorm)
        scale_ref[0] = jnp.sqrt(jnp.abs(cr_t / (cr_n + EPS)))

    o_ref[...] = ynorm * scale_ref[0]


def _finish(acc, cnt, x, wm2, bm2_2d, wu1, bu1_2d, wu2, bu2_2d):
    blk = 1000
    return pl.pallas_call(
        _finish_body,
        grid=(N // blk,),
        in_specs=[
            pl.BlockSpec((NC, blk, F), lambda i: (0, i, 0)),
            pl.BlockSpec((NC, blk, 16), lambda i: (0, i, 0)),
            pl.BlockSpec((blk, F), lambda i: (i, 0)),
            pl.BlockSpec((F, F), lambda i: (0, 0)),
            pl.BlockSpec((1, F), lambda i: (0, 0)),
            pl.BlockSpec((2 * F, F), lambda i: (0, 0)),
            pl.BlockSpec((1, F), lambda i: (0, 0)),
            pl.BlockSpec((F, F), lambda i: (0, 0)),
            pl.BlockSpec((1, F), lambda i: (0, 0)),
        ],
        out_specs=pl.BlockSpec((blk, F), lambda i: (i, 0)),
        out_shape=jax.ShapeDtypeStruct((N, F), jnp.float32),
        scratch_shapes=[pltpu.SMEM((1,), jnp.float32)],
    )(acc, cnt, x, wm2, bm2_2d, wu1, bu1_2d, wu2, bu2_2d)


def kernel(x, edge_index, edge_attr, Wm1, bm1, Wm2, bm2, Wu1, bu1, Wu2, bu2):
    xw = _matmul_bias(x, Wm1[:F], jnp.zeros((1, F), jnp.float32), 1000)
    eaw = _matmul_bias(edge_attr, Wm1[F:], bm1.reshape(1, F), 2000)
    col2 = edge_index[1].reshape(NW, NQ, CPQ, CB)
    row2 = edge_index[0].reshape(NW, NQ, CPQ, CB)
    acc, cnt = _make_sc_call()(eaw, xw, col2, row2)
    return _finish(acc, cnt, x, Wm2[_ACC_PERM], bm2.reshape(1, F), Wu1,
                   bu1.reshape(1, F), Wu2, bu2.reshape(1, F))


# speedup vs baseline: 1.6892x; 1.6892x over previous
import functools

import jax
import jax.numpy as jnp
import numpy as np
from jax import lax
from jax.experimental import pallas as pl
from jax.experimental.pallas import tpu as pltpu
from jax.experimental.pallas import tpu_sc as plsc

N = 10000
E = 320000
F = 128
EPS = 1e-8

NC, NS = 2, 16
NW = NC * NS
EPW = E // NW
CB = 40
NBLK = EPW // CB
NQ = 5
CPQ = NBLK // NQ
N_PAD = 10240
RPT = N_PAD // NS

_ACC_PERM = np.concatenate(
    [np.concatenate([32 * m + 2 * np.arange(16),
                     32 * m + 2 * np.arange(16) + 1]) for m in range(4)])


def _mm_body(a_ref, w_ref, b_ref, o_ref):
    o_ref[...] = jnp.dot(a_ref[...].astype(jnp.bfloat16),
                         w_ref[...].astype(jnp.bfloat16),
                         preferred_element_type=jnp.float32) + b_ref[...]


def _matmul_bias(a, w, b2d, blk):
    g = a.shape[0] // blk
    return pl.pallas_call(
        _mm_body,
        grid=(g,),
        in_specs=[
            pl.BlockSpec((blk, a.shape[1]), lambda i: (i, 0)),
            pl.BlockSpec(w.shape, lambda i: (0, 0)),
            pl.BlockSpec((1, b2d.shape[1]), lambda i: (0, 0)),
        ],
        out_specs=pl.BlockSpec((blk, w.shape[1]), lambda i: (i, 0)),
        out_shape=jax.ShapeDtypeStruct((a.shape[0], w.shape[1]), jnp.float32),
    )(a, w, b2d)


def _sc_body(eaw_hbm, xw_hbm, col_hbm, row_hbm, acc_out, cnt_out,
             col_v, row_v, ea_v0, ea_v1, xg_v0, xg_v1, h_v0, h_v1,
             ones_v, zbs_v, acc_sh, cnt_sh,
             sem_ld0, sem_ld1, sem_sc0, sem_sc1):
    cid = lax.axis_index("c")
    sid = lax.axis_index("s")
    wid = cid * NS + sid

    ea_v = (ea_v0, ea_v1)
    xg_v = (xg_v0, xg_v1)
    h_v = (h_v0, h_v1)
    sem_ld = (sem_ld0, sem_ld1)
    sem_sc = (sem_sc0, sem_sc1)

    zf = jnp.zeros((16,), jnp.float32)
    of = jnp.ones((16,), jnp.float32)

    def zero_h(i, _):
        for k in range(F // 16):
            h_v0[i, pl.ds(k * 16, 16)] = zf
        return 0
    lax.fori_loop(0, CB, zero_h, 0)

    def fill_ones(i, _):
        ones_v[i, :] = of
        zbs_v[i, :] = zf
        return 0
    lax.fori_loop(0, CB, fill_ones, 0)

    for m in range(RPT // CB):
        pltpu.sync_copy(h_v0, acc_sh.at[pl.ds(sid * RPT + m * CB, CB)])
        pltpu.sync_copy(zbs_v, cnt_sh.at[pl.ds(sid * RPT + m * CB, CB)])
    plsc.subcore_barrier()

    def wait_load(p):
        pltpu.make_async_copy(eaw_hbm.at[pl.ds(0, CB)], ea_v[p], sem_ld[p]).wait()
        pltpu.make_async_copy(xw_hbm.at[col_v.at[0]], xg_v[p], sem_ld[p]).wait()

    def wait_scatter(p):
        pltpu.make_async_copy(h_v[p], acc_sh.at[row_v.at[0]], sem_sc[p]).wait()
        pltpu.make_async_copy(ones_v, cnt_sh.at[row_v.at[0]], sem_sc[p]).wait()

    for q in range(NQ):
        pltpu.sync_copy(col_hbm.at[wid, q], col_v)
        pltpu.sync_copy(row_hbm.at[wid, q], row_v)

        def start_load(j, p):
            e0 = wid * EPW + (q * CPQ + j) * CB
            pltpu.async_copy(eaw_hbm.at[pl.ds(e0, CB)], ea_v[p], sem_ld[p])
            pltpu.async_copy(xw_hbm.at[col_v.at[j]], xg_v[p], sem_ld[p])

        def process(j, m, p):
            wait_load(p)

            @pl.when(m > 0)
            def _():
                wait_scatter(p)

            def rowfn(i, _):
                for k in range(F // 16):
                    s = pl.ds(k * 16, 16)
                    h_v[p][i, s] = jnp.maximum(xg_v[p][i, s] + ea_v[p][i, s],
                                               0.0)
                return 0
            lax.fori_loop(0, CB, rowfn, 0)
            pltpu.async_copy(h_v[p], acc_sh.at[row_v.at[j]], sem_sc[p],
                             add=True)
            pltpu.async_copy(ones_v, cnt_sh.at[row_v.at[j]], sem_sc[p],
                             add=True)

        start_load(0, 0)

        def pair(m, _):
            start_load(2 * m + 1, 1)
            process(2 * m, m, 0)

            @pl.when(m < CPQ // 2 - 1)
            def _():
                start_load(2 * m + 2, 0)
            process(2 * m + 1, m, 1)
            return 0
        lax.fori_loop(0, CPQ // 2, pair, 0)
        wait_scatter(0)
        wait_scatter(1)

    plsc.subcore_barrier()
    pltpu.sync_copy(acc_sh.at[pl.ds(sid * RPT, RPT)],
                    acc_out.at[cid, pl.ds(sid * RPT, RPT)])
    pltpu.sync_copy(cnt_sh.at[pl.ds(sid * RPT, RPT)],
                    cnt_out.at[cid, pl.ds(sid * RPT, RPT)])


@functools.lru_cache(maxsize=1)
def _make_sc_call():
  return pl.kernel(
    _sc_body,
    out_type=[
        jax.ShapeDtypeStruct((NC, N_PAD, F), jnp.float32),
        jax.ShapeDtypeStruct((NC, N_PAD, 16), jnp.float32),
    ],
    mesh=plsc.VectorSubcoreMesh(core_axis_name="c", subcore_axis_name="s",
                                num_cores=NC, num_subcores=NS),
    scratch_types=(
        [pltpu.VMEM((CPQ, CB), jnp.int32)] * 2
        + [pltpu.VMEM((CB, F), jnp.float32)] * 6
        + [pltpu.VMEM((CB, 16), jnp.float32),
           pltpu.VMEM((CB, 16), jnp.float32),
           pltpu.VMEM_SHARED((N_PAD, F), jnp.float32),
           pltpu.VMEM_SHARED((N_PAD, 16), jnp.float32)]
        + [pltpu.SemaphoreType.DMA] * 4
    ),
    compiler_params=pltpu.CompilerParams(use_tc_tiling_on_sc=False),
  )


def _cross_ratio(m):
    a, b, c, d = m[0], m[1], m[2], m[3]
    d13 = jnp.sqrt(jnp.sum((a - c) ** 2)) + EPS
    d24 = jnp.sqrt(jnp.sum((b - d) ** 2)) + EPS
    d14 = jnp.sqrt(jnp.sum((a - d) ** 2)) + EPS
    d23 = jnp.sqrt(jnp.sum((b - c) ** 2)) + EPS
    return (d13 * d24) / (d14 * d23)


def _finish_body(acc_ref, cnt_ref, x_ref, wm2_ref, bm2_ref, wu1_ref, bu1_ref,
                 wu2_ref, bu2_ref, o_ref, scale_ref):
    i = pl.program_id(0)
    agg = acc_ref[0] + acc_ref[1]
    cnt = cnt_ref[0, :, 0] + cnt_ref[1, :, 0]
    denom = (cnt + EPS)[:, None]
    mean = (jnp.dot(agg, wm2_ref[...], preferred_element_type=jnp.float32)
            + cnt[:, None] * bm2_ref[...]) / denom
    xb = x_ref[...]
    u = jnp.maximum(
        jnp.dot(xb, wu1_ref[:F], preferred_element_type=jnp.float32)
        + jnp.dot(mean, wu1_ref[F:], preferred_element_type=jnp.float32)
        + bu1_ref[...], 0.0)
    y = jnp.dot(u, wu2_ref[...], preferred_element_type=jnp.float32) + bu2_ref[...]
    nrm = jnp.sqrt(jnp.sum(y * y, axis=1, keepdims=True))
    ynorm = y / (nrm + EPS)

    @pl.when(i == 0)
    def _():
        cr_t = _cross_ratio(xb)
        cr_n = _cross_ratio(ynorm)
        scale_ref[0] = jnp.sqrt(jnp.abs(cr_t / (cr_n + EPS)))

    o_ref[...] = ynorm * scale_ref[0]


def _finish(acc, cnt, x, wm2, bm2_2d, wu1, bu1_2d, wu2, bu2_2d):
    blk = 1000
    return pl.pallas_call(
        _finish_body,
        grid=(N // blk,),
        in_specs=[
            pl.BlockSpec((NC, blk, F), lambda i: (0, i, 0)),
            pl.BlockSpec((NC, blk, 16), lambda i: (0, i, 0)),
            pl.BlockSpec((blk, F), lambda i: (i, 0)),
            pl.BlockSpec((F, F), lambda i: (0, 0)),
            pl.BlockSpec((1, F), lambda i: (0, 0)),
            pl.BlockSpec((2 * F, F), lambda i: (0, 0)),
            pl.BlockSpec((1, F), lambda i: (0, 0)),
            pl.BlockSpec((F, F), lambda i: (0, 0)),
            pl.BlockSpec((1, F), lambda i: (0, 0)),
        ],
        out_specs=pl.BlockSpec((blk, F), lambda i: (i, 0)),
        out_shape=jax.ShapeDtypeStruct((N, F), jnp.float32),
        scratch_shapes=[pltpu.SMEM((1,), jnp.float32)],
    )(acc, cnt, x, wm2, bm2_2d, wu1, bu1_2d, wu2, bu2_2d)


def kernel(x, edge_index, edge_attr, Wm1, bm1, Wm2, bm2, Wu1, bu1, Wu2, bu2):
    xw = _matmul_bias(x, Wm1[:F], jnp.zeros((1, F), jnp.float32), 1000)
    eaw = _matmul_bias(edge_attr, Wm1[F:], bm1.reshape(1, F), 2000)
    col2 = edge_index[1].reshape(NW, NQ, CPQ, CB)
    row2 = edge_index[0].reshape(NW, NQ, CPQ, CB)
    acc, cnt = _make_sc_call()(eaw, xw, col2, row2)
    return _finish(acc, cnt, x, Wm2, bm2.reshape(1, F), Wu1,
                   bu1.reshape(1, F), Wu2, bu2.reshape(1, F))


# speedup vs baseline: 2.0283x; 1.2007x over previous
import functools

import jax
import jax.numpy as jnp
import numpy as np
from jax import lax
from jax.experimental import pallas as pl
from jax.experimental.pallas import tpu as pltpu
from jax.experimental.pallas import tpu_sc as plsc

N = 10000
E = 320000
F = 128
EPS = 1e-8

NC, NS = 2, 16
NW = NC * NS
EPW = E // NW
CB = 40
NBLK = EPW // CB
NQ = 5
CPQ = NBLK // NQ
N_PAD = 10240
RPT = N_PAD // NS

_ACC_PERM = np.concatenate(
    [np.concatenate([32 * m + 2 * np.arange(16),
                     32 * m + 2 * np.arange(16) + 1]) for m in range(4)])


def _mm_body(a_ref, w_ref, b_ref, o_ref):
    o_ref[...] = jnp.dot(a_ref[...].astype(jnp.bfloat16),
                         w_ref[...].astype(jnp.bfloat16),
                         preferred_element_type=jnp.float32) + b_ref[...]


def _matmul_bias(a, w, b2d, blk):
    g = a.shape[0] // blk
    return pl.pallas_call(
        _mm_body,
        grid=(g,),
        in_specs=[
            pl.BlockSpec((blk, a.shape[1]), lambda i: (i, 0)),
            pl.BlockSpec(w.shape, lambda i: (0, 0)),
            pl.BlockSpec((1, b2d.shape[1]), lambda i: (0, 0)),
        ],
        out_specs=pl.BlockSpec((blk, w.shape[1]), lambda i: (i, 0)),
        out_shape=jax.ShapeDtypeStruct((a.shape[0], w.shape[1]), jnp.float32),
    )(a, w, b2d)


def _sc_body(eaw_hbm, xw_hbm, col_hbm, row_hbm, acc_out, cnt_out,
             col_v, row_v, ea_v0, ea_v1, xg_v0, xg_v1, h_v0, h_v1,
             ones_v, zbs_v, acc_sh, cnt_sh,
             sem_ld0, sem_ld1, sem_sc0, sem_sc1):
    cid = lax.axis_index("c")
    sid = lax.axis_index("s")
    wid = cid * NS + sid

    ea_v = (ea_v0, ea_v1)
    xg_v = (xg_v0, xg_v1)
    h_v = (h_v0, h_v1)
    sem_ld = (sem_ld0, sem_ld1)
    sem_sc = (sem_sc0, sem_sc1)

    zf = jnp.zeros((16,), jnp.float32)
    of = jnp.ones((16,), jnp.float32)

    def zero_h(i, _):
        for k in range(F // 16):
            h_v0[i, pl.ds(k * 16, 16)] = zf
        return 0
    lax.fori_loop(0, CB, zero_h, 0)

    def fill_ones(i, _):
        ones_v[i, :] = of
        zbs_v[i, :] = zf
        return 0
    lax.fori_loop(0, CB, fill_ones, 0)

    for m in range(RPT // CB):
        pltpu.sync_copy(h_v0, acc_sh.at[pl.ds(sid * RPT + m * CB, CB)])
        pltpu.sync_copy(zbs_v, cnt_sh.at[pl.ds(sid * RPT + m * CB, CB)])
    plsc.subcore_barrier()

    def wait_load(p):
        pltpu.make_async_copy(eaw_hbm.at[pl.ds(0, CB)], ea_v[p], sem_ld[p]).wait()
        pltpu.make_async_copy(xw_hbm.at[col_v.at[0]], xg_v[p], sem_ld[p]).wait()

    def wait_scatter(p):
        pltpu.make_async_copy(h_v[p], acc_sh.at[row_v.at[0]], sem_sc[p]).wait()
        pltpu.make_async_copy(ones_v, cnt_sh.at[row_v.at[0]], sem_sc[p]).wait()

    for q in range(NQ):
        pltpu.sync_copy(col_hbm.at[wid, q], col_v)
        pltpu.sync_copy(row_hbm.at[wid, q], row_v)

        def start_load(j, p):
            e0 = wid * EPW + (q * CPQ + j) * CB
            pltpu.async_copy(eaw_hbm.at[pl.ds(e0, CB)], ea_v[p], sem_ld[p])
            pltpu.async_copy(xw_hbm.at[col_v.at[j]], xg_v[p], sem_ld[p])

        def process(j, m, p):
            wait_load(p)

            @pl.when(m > 0)
            def _():
                wait_scatter(p)

            def rowfn(i, _):
                for k in range(F // 16):
                    s = pl.ds(k * 16, 16)
                    h_v[p][i, s] = jnp.maximum(xg_v[p][i, s] + ea_v[p][i, s],
                                               0.0)
                return 0
            lax.fori_loop(0, CB, rowfn, 0)
            pltpu.async_copy(h_v[p], acc_sh.at[row_v.at[j]], sem_sc[p],
                             add=True)
            pltpu.async_copy(ones_v, cnt_sh.at[row_v.at[j]], sem_sc[p],
                             add=True)

        start_load(0, 0)

        def pair(m, _):
            start_load(2 * m + 1, 1)
            process(2 * m, m, 0)

            @pl.when(m < CPQ // 2 - 1)
            def _():
                start_load(2 * m + 2, 0)
            process(2 * m + 1, m, 1)
            return 0
        lax.fori_loop(0, CPQ // 2, pair, 0)
        wait_scatter(0)
        wait_scatter(1)

    plsc.subcore_barrier()
    pltpu.sync_copy(acc_sh.at[pl.ds(sid * RPT, RPT)],
                    acc_out.at[cid, pl.ds(sid * RPT, RPT)])
    pltpu.sync_copy(cnt_sh.at[pl.ds(sid * RPT, RPT)],
                    cnt_out.at[cid, pl.ds(sid * RPT, RPT)])


@functools.lru_cache(maxsize=1)
def _make_sc_call():
  return pl.kernel(
    _sc_body,
    out_type=[
        jax.ShapeDtypeStruct((NC, N_PAD, F), jnp.float32),
        jax.ShapeDtypeStruct((NC, N_PAD, 16), jnp.float32),
    ],
    mesh=plsc.VectorSubcoreMesh(core_axis_name="c", subcore_axis_name="s",
                                num_cores=NC, num_subcores=NS),
    scratch_types=(
        [pltpu.VMEM((CPQ, CB), jnp.int32)] * 2
        + [pltpu.VMEM((CB, F), jnp.float32)] * 6
        + [pltpu.VMEM((CB, 16), jnp.float32),
           pltpu.VMEM((CB, 16), jnp.float32),
           pltpu.VMEM_SHARED((N_PAD, F), jnp.float32),
           pltpu.VMEM_SHARED((N_PAD, 16), jnp.float32)]
        + [pltpu.SemaphoreType.DMA] * 4
    ),
    compiler_params=pltpu.CompilerParams(use_tc_tiling_on_sc=False),
  )


def _cross_ratio(m):
    a, b, c, d = m[0], m[1], m[2], m[3]
    d13 = jnp.sqrt(jnp.sum((a - c) ** 2)) + EPS
    d24 = jnp.sqrt(jnp.sum((b - d) ** 2)) + EPS
    d14 = jnp.sqrt(jnp.sum((a - d) ** 2)) + EPS
    d23 = jnp.sqrt(jnp.sum((b - c) ** 2)) + EPS
    return (d13 * d24) / (d14 * d23)


def _finish_body(acc_ref, cnt_ref, x_ref, wm2_ref, bm2_ref, wu1_ref, bu1_ref,
                 wu2_ref, bu2_ref, o_ref, scale_ref):
    i = pl.program_id(0)
    agg = acc_ref[0] + acc_ref[1]
    cnt = cnt_ref[0, :, 0] + cnt_ref[1, :, 0]
    denom = (cnt + EPS)[:, None]
    mean = (jnp.dot(agg, wm2_ref[...], preferred_element_type=jnp.float32)
            + cnt[:, None] * bm2_ref[...]) / denom
    xb = x_ref[...]
    u = jnp.maximum(
        jnp.dot(xb, wu1_ref[:F], preferred_element_type=jnp.float32)
        + jnp.dot(mean, wu1_ref[F:], preferred_element_type=jnp.float32)
        + bu1_ref[...], 0.0)
    y = jnp.dot(u, wu2_ref[...], preferred_element_type=jnp.float32) + bu2_ref[...]
    nrm = jnp.sqrt(jnp.sum(y * y, axis=1, keepdims=True))
    ynorm = y / (nrm + EPS)

    @pl.when(i == 0)
    def _():
        cr_t = _cross_ratio(xb)
        cr_n = _cross_ratio(ynorm)
        scale_ref[0] = jnp.sqrt(jnp.abs(cr_t / (cr_n + EPS)))

    o_ref[...] = ynorm * scale_ref[0]


def _finish(acc, cnt, x, wm2, bm2_2d, wu1, bu1_2d, wu2, bu2_2d):
    blk = 1000
    return pl.pallas_call(
        _finish_body,
        grid=(N // blk,),
        in_specs=[
            pl.BlockSpec((NC, blk, F), lambda i: (0, i, 0)),
            pl.BlockSpec((NC, blk, 16), lambda i: (0, i, 0)),
            pl.BlockSpec((blk, F), lambda i: (i, 0)),
            pl.BlockSpec((F, F), lambda i: (0, 0)),
            pl.BlockSpec((1, F), lambda i: (0, 0)),
            pl.BlockSpec((2 * F, F), lambda i: (0, 0)),
            pl.BlockSpec((1, F), lambda i: (0, 0)),
            pl.BlockSpec((F, F), lambda i: (0, 0)),
            pl.BlockSpec((1, F), lambda i: (0, 0)),
        ],
        out_specs=pl.BlockSpec((blk, F), lambda i: (i, 0)),
        out_shape=jax.ShapeDtypeStruct((N, F), jnp.float32),
        scratch_shapes=[pltpu.SMEM((1,), jnp.float32)],
    )(acc, cnt, x, wm2, bm2_2d, wu1, bu1_2d, wu2, bu2_2d)


def kernel(x, edge_index, edge_attr, Wm1, bm1, Wm2, bm2, Wu1, bu1, Wu2, bu2):
    xw = _matmul_bias(x, Wm1[:F], jnp.zeros((1, F), jnp.float32), 1000)
    eaw = _matmul_bias(edge_attr, Wm1[F:], bm1.reshape(1, F), 8000)
    col2 = edge_index[1].reshape(NW, NQ, CPQ, CB)
    row2 = edge_index[0].reshape(NW, NQ, CPQ, CB)
    acc, cnt = _make_sc_call()(eaw, xw, col2, row2)
    return _finish(acc, cnt, x, Wm2, bm2.reshape(1, F), Wu1,
                   bu1.reshape(1, F), Wu2, bu2.reshape(1, F))


# speedup vs baseline: 2.0362x; 1.0039x over previous
import functools

import jax
import jax.numpy as jnp
import numpy as np
from jax import lax
from jax.experimental import pallas as pl
from jax.experimental.pallas import tpu as pltpu
from jax.experimental.pallas import tpu_sc as plsc

N = 10000
E = 320000
F = 128
EPS = 1e-8

NC, NS = 2, 16
NW = NC * NS
EPW = E // NW
CB = 40
NBLK = EPW // CB
NQ = 5
CPQ = NBLK // NQ
N_PAD = 10240
RPT = N_PAD // NS

_ACC_PERM = np.concatenate(
    [np.concatenate([32 * m + 2 * np.arange(16),
                     32 * m + 2 * np.arange(16) + 1]) for m in range(4)])


def _mm_body(a_ref, w_ref, b_ref, o_ref):
    o_ref[...] = jnp.dot(a_ref[...].astype(jnp.bfloat16),
                         w_ref[...].astype(jnp.bfloat16),
                         preferred_element_type=jnp.float32) + b_ref[...]


def _matmul_bias(a, w, b2d, blk):
    g = a.shape[0] // blk
    return pl.pallas_call(
        _mm_body,
        grid=(g,),
        in_specs=[
            pl.BlockSpec((blk, a.shape[1]), lambda i: (i, 0)),
            pl.BlockSpec(w.shape, lambda i: (0, 0)),
            pl.BlockSpec((1, b2d.shape[1]), lambda i: (0, 0)),
        ],
        out_specs=pl.BlockSpec((blk, w.shape[1]), lambda i: (i, 0)),
        out_shape=jax.ShapeDtypeStruct((a.shape[0], w.shape[1]), jnp.float32),
    )(a, w, b2d)


def _sc_body(eaw_hbm, xw_hbm, col_hbm, row_hbm, acc_out, cnt_out,
             col_v, row_v, ea_v0, ea_v1, xg_v0, xg_v1, h_v0, h_v1,
             ones_v, zbs_v, acc_sh, cnt_sh,
             sem_ld0, sem_ld1, sem_sc0, sem_sc1):
    cid = lax.axis_index("c")
    sid = lax.axis_index("s")
    wid = cid * NS + sid

    ea_v = (ea_v0, ea_v1)
    xg_v = (xg_v0, xg_v1)
    h_v = (h_v0, h_v1)
    sem_ld = (sem_ld0, sem_ld1)
    sem_sc = (sem_sc0, sem_sc1)

    zf = jnp.zeros((16,), jnp.float32)
    of = jnp.ones((16,), jnp.float32)

    def zero_h(i, _):
        for k in range(F // 16):
            h_v0[i, pl.ds(k * 16, 16)] = zf
        return 0
    lax.fori_loop(0, CB, zero_h, 0)

    def fill_ones(i, _):
        ones_v[i, :] = of
        zbs_v[i, :] = zf
        return 0
    lax.fori_loop(0, CB, fill_ones, 0)

    for m in range(RPT // CB):
        pltpu.sync_copy(h_v0, acc_sh.at[pl.ds(sid * RPT + m * CB, CB)])
        pltpu.sync_copy(zbs_v, cnt_sh.at[pl.ds(sid * RPT + m * CB, CB)])
    plsc.subcore_barrier()

    def wait_load(p):
        pltpu.make_async_copy(eaw_hbm.at[pl.ds(0, CB)], ea_v[p], sem_ld[p]).wait()
        pltpu.make_async_copy(xw_hbm.at[col_v.at[0]], xg_v[p], sem_ld[p]).wait()

    def wait_scatter(p):
        pltpu.make_async_copy(h_v[p], acc_sh.at[row_v.at[0]], sem_sc[p]).wait()
        pltpu.make_async_copy(ones_v, cnt_sh.at[row_v.at[0]], sem_sc[p]).wait()

    for q in range(NQ):
        pltpu.sync_copy(col_hbm.at[wid, q], col_v)
        pltpu.sync_copy(row_hbm.at[wid, q], row_v)

        def start_load(j, p):
            e0 = wid * EPW + (q * CPQ + j) * CB
            pltpu.async_copy(eaw_hbm.at[pl.ds(e0, CB)], ea_v[p], sem_ld[p])
            pltpu.async_copy(xw_hbm.at[col_v.at[j]], xg_v[p], sem_ld[p])

        def process(j, m, p):
            wait_load(p)

            @pl.when(m > 0)
            def _():
                wait_scatter(p)

            def rowfn(i, _):
                for k in range(F // 16):
                    s = pl.ds(k * 16, 16)
                    h_v[p][i, s] = jnp.maximum(xg_v[p][i, s] + ea_v[p][i, s],
                                               0.0)
                return 0
            lax.fori_loop(0, CB, rowfn, 0)
            pltpu.async_copy(h_v[p], acc_sh.at[row_v.at[j]], sem_sc[p],
                             add=True)
            pltpu.async_copy(ones_v, cnt_sh.at[row_v.at[j]], sem_sc[p],
                             add=True)

        start_load(0, 0)

        def pair(m, _):
            start_load(2 * m + 1, 1)
            process(2 * m, m, 0)

            @pl.when(m < CPQ // 2 - 1)
            def _():
                start_load(2 * m + 2, 0)
            process(2 * m + 1, m, 1)
            return 0
        lax.fori_loop(0, CPQ // 2, pair, 0)
        wait_scatter(0)
        wait_scatter(1)

    plsc.subcore_barrier()
    pltpu.sync_copy(acc_sh.at[pl.ds(sid * RPT, RPT)],
                    acc_out.at[cid, pl.ds(sid * RPT, RPT)])
    pltpu.sync_copy(cnt_sh.at[pl.ds(sid * RPT, RPT)],
                    cnt_out.at[cid, pl.ds(sid * RPT, RPT)])


@functools.lru_cache(maxsize=1)
def _make_sc_call():
  return pl.kernel(
    _sc_body,
    out_type=[
        jax.ShapeDtypeStruct((NC, N_PAD, F), jnp.float32),
        jax.ShapeDtypeStruct((NC, N_PAD, 16), jnp.float32),
    ],
    mesh=plsc.VectorSubcoreMesh(core_axis_name="c", subcore_axis_name="s",
                                num_cores=NC, num_subcores=NS),
    scratch_types=(
        [pltpu.VMEM((CPQ, CB), jnp.int32)] * 2
        + [pltpu.VMEM((CB, F), jnp.float32)] * 6
        + [pltpu.VMEM((CB, 16), jnp.float32),
           pltpu.VMEM((CB, 16), jnp.float32),
           pltpu.VMEM_SHARED((N_PAD, F), jnp.float32),
           pltpu.VMEM_SHARED((N_PAD, 16), jnp.float32)]
        + [pltpu.SemaphoreType.DMA] * 4
    ),
    compiler_params=pltpu.CompilerParams(use_tc_tiling_on_sc=False),
  )


def _cross_ratio(m):
    a, b, c, d = m[0], m[1], m[2], m[3]
    d13 = jnp.sqrt(jnp.sum((a - c) ** 2)) + EPS
    d24 = jnp.sqrt(jnp.sum((b - d) ** 2)) + EPS
    d14 = jnp.sqrt(jnp.sum((a - d) ** 2)) + EPS
    d23 = jnp.sqrt(jnp.sum((b - c) ** 2)) + EPS
    return (d13 * d24) / (d14 * d23)


def _finish_body(acc_ref, cnt_ref, x_ref, wm2_ref, bm2_ref, wu1_ref, bu1_ref,
                 wu2_ref, bu2_ref, o_ref, scale_ref):
    i = pl.program_id(0)
    agg = acc_ref[0] + acc_ref[1]
    cnt = cnt_ref[0, :, 0] + cnt_ref[1, :, 0]
    denom = (cnt + EPS)[:, None]
    mean = (jnp.dot(agg, wm2_ref[...], preferred_element_type=jnp.float32)
            + cnt[:, None] * bm2_ref[...]) / denom
    xb = x_ref[...]
    u = jnp.maximum(
        jnp.dot(xb, wu1_ref[:F], preferred_element_type=jnp.float32)
        + jnp.dot(mean, wu1_ref[F:], preferred_element_type=jnp.float32)
        + bu1_ref[...], 0.0)
    y = jnp.dot(u, wu2_ref[...], preferred_element_type=jnp.float32) + bu2_ref[...]
    nrm = jnp.sqrt(jnp.sum(y * y, axis=1, keepdims=True))
    ynorm = y / (nrm + EPS)

    @pl.when(i == 0)
    def _():
        cr_t = _cross_ratio(xb)
        cr_n = _cross_ratio(ynorm)
        scale_ref[0] = jnp.sqrt(jnp.abs(cr_t / (cr_n + EPS)))

    o_ref[...] = ynorm * scale_ref[0]


def _finish(acc, cnt, x, wm2, bm2_2d, wu1, bu1_2d, wu2, bu2_2d):
    blk = 1000
    return pl.pallas_call(
        _finish_body,
        grid=(N // blk,),
        in_specs=[
            pl.BlockSpec((NC, blk, F), lambda i: (0, i, 0)),
            pl.BlockSpec((NC, blk, 16), lambda i: (0, i, 0)),
            pl.BlockSpec((blk, F), lambda i: (i, 0)),
            pl.BlockSpec((F, F), lambda i: (0, 0)),
            pl.BlockSpec((1, F), lambda i: (0, 0)),
            pl.BlockSpec((2 * F, F), lambda i: (0, 0)),
            pl.BlockSpec((1, F), lambda i: (0, 0)),
            pl.BlockSpec((F, F), lambda i: (0, 0)),
            pl.BlockSpec((1, F), lambda i: (0, 0)),
        ],
        out_specs=pl.BlockSpec((blk, F), lambda i: (i, 0)),
        out_shape=jax.ShapeDtypeStruct((N, F), jnp.float32),
        scratch_shapes=[pltpu.SMEM((1,), jnp.float32)],
    )(acc, cnt, x, wm2, bm2_2d, wu1, bu1_2d, wu2, bu2_2d)


def kernel(x, edge_index, edge_attr, Wm1, bm1, Wm2, bm2, Wu1, bu1, Wu2, bu2):
    xw = _matmul_bias(x, Wm1[:F], jnp.zeros((1, F), jnp.float32), 1000)
    eaw = _matmul_bias(edge_attr, Wm1[F:], bm1.reshape(1, F), 16000)
    col2 = edge_index[1].reshape(NW, NQ, CPQ, CB)
    row2 = edge_index[0].reshape(NW, NQ, CPQ, CB)
    acc, cnt = _make_sc_call()(eaw, xw, col2, row2)
    return _finish(acc, cnt, x, Wm2, bm2.reshape(1, F), Wu1,
                   bu1.reshape(1, F), Wu2, bu2.reshape(1, F))
